# Initial kernel scaffold; baseline (speedup 1.0000x reference)
import functools

import jax
import jax.numpy as jnp
from jax import lax
from jax.experimental import pallas as pl
from jax.experimental.pallas import tpu as pltpu
from jax.experimental.pallas import tpu_sc as plsc

_N = 10000
_E = 160000
_D = 256
_DH = 128
_DA = _DH + 16
_C = 128
_NTILES = 16
_NCHUNKS = _E // _C
_CPT = -(-_NCHUNKS // _NTILES)
_NPT = _N // _NTILES
_ZR = 125

_HIGH = lax.Precision.HIGHEST


def _sc_segment_sums(x0, x1, src, dst):
    mesh = plsc.VectorSubcoreMesh(core_axis_name="c", subcore_axis_name="s")
    out = jax.ShapeDtypeStruct((_N, _DA), jnp.float32)

    @functools.partial(
        pl.kernel,
        mesh=mesh,
        out_type=(out, out, out, out),
        scratch_types=[
            pltpu.VMEM_SHARED((_N, _DA), jnp.float32),
            pltpu.VMEM((_C, _DA), jnp.float32),
            pltpu.VMEM((_C,), jnp.int32),
            pltpu.VMEM((_C,), jnp.int32),
            pltpu.VMEM((_ZR, _DA), jnp.float32),
            pltpu.SemaphoreType.DMA,
        ],
    )
    def k(x0_hbm, x1_hbm, src_hbm, dst_hbm,
          so0_hbm, so1_hbm, si0_hbm, si1_hbm,
          acc, rows, gidx, sidx, zbuf, sem):
        c = lax.axis_index("c")
        s = lax.axis_index("s")
        nbase = s * _NPT

        zero16 = jnp.zeros((16,), jnp.float32)

        def _zrow(r, carry):
            def _zcol(j, carry2):
                zbuf[r, pl.ds(j * 16, 16)] = zero16
                return carry2
            return lax.fori_loop(0, _DA // 16, _zcol, carry)

        lax.fori_loop(0, _ZR, _zrow, 0)

        def _zero_acc():
            def _z(i, carry):
                pltpu.sync_copy(zbuf, acc.at[pl.ds(nbase + i * _ZR, _ZR)])
                return carry
            lax.fori_loop(0, _NPT // _ZR, _z, 0)

        def _direction(g_hbm, s_hbm, x_hbm, out_hbm):
            _zero_acc()
            plsc.subcore_barrier()

            def _chunk(kk, carry):
                j = kk * _NTILES + s
                @pl.when(j < _NCHUNKS)
                def _():
                    off = j * _C
                    pltpu.sync_copy(g_hbm.at[pl.ds(off, _C)], gidx)
                    pltpu.sync_copy(s_hbm.at[pl.ds(off, _C)], sidx)
                    pltpu.async_copy(x_hbm.at[gidx], rows, sem).wait()
                    pltpu.sync_copy(rows, acc.at[sidx], add=True)
                return carry

            lax.fori_loop(0, _CPT, _chunk, 0)
            plsc.subcore_barrier()
            pltpu.sync_copy(acc.at[pl.ds(nbase, _NPT)],
                            out_hbm.at[pl.ds(nbase, _NPT)])

        @pl.when(c == 0)
        def _():
            _direction(src_hbm, dst_hbm, x0_hbm, so0_hbm)
            plsc.subcore_barrier()
            _direction(dst_hbm, src_hbm, x0_hbm, si0_hbm)

        @pl.when(c == 1)
        def _():
            _direction(src_hbm, dst_hbm, x1_hbm, so1_hbm)
            plsc.subcore_barrier()
            _direction(dst_hbm, src_hbm, x1_hbm, si1_hbm)

    return k(x0, x1, src, dst)


_BLK = 2000


def _tc_body(x_ref, so0_ref, so1_ref, si0_ref, si1_ref, rf_ref,
             wo_ref, bo_ref, wi_ref, bi_ref, ws_ref, bs_ref, wr_ref, br_ref,
             out_ref, rout_ref):
    f32 = jnp.float32
    dn_t = (((1,), (1,)), ((), ()))
    dn_n = (((1,), (0,)), ((), ()))

    wo = wo_ref[...]
    wi = wi_ref[...]
    ws = ws_ref[...]

    acc = lax.dot_general(x_ref[...], ws, dn_t,
                          preferred_element_type=f32, precision=_HIGH)
    acc += lax.dot_general(so0_ref[:, :_DH], wo[:, :_DH], dn_t,
                           preferred_element_type=f32, precision=_HIGH)
    acc += lax.dot_general(so1_ref[:, :_DH], wo[:, _DH:], dn_t,
                           preferred_element_type=f32, precision=_HIGH)
    acc += lax.dot_general(si0_ref[:, :_DH], wi[:, :_DH], dn_t,
                           preferred_element_type=f32, precision=_HIGH)
    acc += lax.dot_general(si1_ref[:, :_DH], wi[:, _DH:], dn_t,
                           preferred_element_type=f32, precision=_HIGH)

    rf = rf_ref[...]
    r_wo = lax.dot_general(rf, wo, dn_t, preferred_element_type=f32,
                           precision=_HIGH)
    r_wi = lax.dot_general(rf, wi, dn_t, preferred_element_type=f32,
                           precision=_HIGH)
    r_ws = lax.dot_general(rf, ws, dn_t, preferred_element_type=f32,
                           precision=_HIGH)
    c_o = bo_ref[...] - r_wo[0:1, :]
    c_i = bi_ref[...] - r_wi[1:2, :]
    c_s = bs_ref[...] - r_ws[2:3, :]

    acc += lax.dot_general(so0_ref[:, _DH:], jnp.broadcast_to(c_o / 16.0, (16, _D)),
                           dn_n, preferred_element_type=f32, precision=_HIGH)
    acc += lax.dot_general(si0_ref[:, _DH:], jnp.broadcast_to(c_i / 16.0, (16, _D)),
                           dn_n, preferred_element_type=f32, precision=_HIGH)
    acc += jnp.broadcast_to(c_s, acc.shape)
    out_ref[...] = acc

    @pl.when(pl.program_id(0) == 0)
    def _():
        r_wr = lax.dot_general(rf, wr_ref[...], dn_t,
                               preferred_element_type=f32, precision=_HIGH)
        rout_ref[...] = r_wr + br_ref[...]


def _tc_combine(x, so0, so1, si0, si1, rf8, W_O, b_O, W_I, b_I, W_S, b_S,
                W_R, b_R):
    rows = lambda i: (i, 0)
    full = lambda i: (0, 0)
    grid = (_N // _BLK,)
    in_specs = [
        pl.BlockSpec((_BLK, _D), rows),
        pl.BlockSpec((_BLK, _DA), rows),
        pl.BlockSpec((_BLK, _DA), rows),
        pl.BlockSpec((_BLK, _DA), rows),
        pl.BlockSpec((_BLK, _DA), rows),
        pl.BlockSpec((8, _D), full),
        pl.BlockSpec((_D, _D), full),
        pl.BlockSpec((1, _D), full),
        pl.BlockSpec((_D, _D), full),
        pl.BlockSpec((1, _D), full),
        pl.BlockSpec((_D, _D), full),
        pl.BlockSpec((1, _D), full),
        pl.BlockSpec((_D, _D), full),
        pl.BlockSpec((1, _D), full),
    ]
    out_specs = (pl.BlockSpec((_BLK, _D), rows), pl.BlockSpec((8, _D), full))
    out_shape = (jax.ShapeDtypeStruct((_N, _D), jnp.float32),
                 jax.ShapeDtypeStruct((8, _D), jnp.float32))
    return pl.pallas_call(
        _tc_body, grid=grid, in_specs=in_specs, out_specs=out_specs,
        out_shape=out_shape,
    )(x, so0, so1, si0, si1, rf8, W_O, b_O, W_I, b_I, W_S, b_S, W_R, b_R)


def kernel(x, edge_index, r_feats, W_O, b_O, W_I, b_I, W_S, b_S, W_R, b_R):
    ones = jnp.ones((_N, 16), jnp.float32)
    x0 = jnp.concatenate([x[:, :_DH], ones], axis=1)
    x1 = jnp.concatenate([x[:, _DH:], ones], axis=1)
    src = edge_index[0]
    dst = edge_index[1]

    so0, so1, si0, si1 = _sc_segment_sums(x0, x1, src, dst)

    rf8 = jnp.pad(r_feats, ((0, 5), (0, 0)))
    n_out, r8 = _tc_combine(
        x, so0, so1, si0, si1, rf8,
        W_O, b_O.reshape(1, _D), W_I, b_I.reshape(1, _D),
        W_S, b_S.reshape(1, _D), W_R, b_R.reshape(1, _D))
    return (n_out, r8[:3])


# speedup vs baseline: 2.7836x; 2.7836x over previous
import functools

import jax
import jax.numpy as jnp
from jax import lax
from jax.experimental import pallas as pl
from jax.experimental.pallas import tpu as pltpu
from jax.experimental.pallas import tpu_sc as plsc

_N = 10000
_E = 160000
_D = 256
_DH = 128
_DA = _DH + 16
_C = 128
_NTILES = 16
_NCHUNKS = _E // _C
_CPT = -(-_NCHUNKS // _NTILES)
_SLAB = 632
_NPAD = _NTILES * _SLAB
_LAST = _N - 15 * _SLAB

_HIGH = lax.Precision.HIGHEST


def _sc_segment_sums(x0, x1, src, dst):
    mesh = plsc.VectorSubcoreMesh(core_axis_name="c", subcore_axis_name="s")
    out = jax.ShapeDtypeStruct((_N, _DA), jnp.float32)

    @functools.partial(
        pl.kernel,
        mesh=mesh,
        out_type=(out, out, out, out),
        compiler_params=pltpu.CompilerParams(use_tc_tiling_on_sc=False),
        scratch_types=[
            pltpu.VMEM_SHARED((_NPAD, _DA), jnp.float32),
            pltpu.VMEM((_C, _DA), jnp.float32),
            pltpu.VMEM((_C,), jnp.int32),
            pltpu.VMEM((_C,), jnp.int32),
            pltpu.SemaphoreType.DMA,
        ],
    )
    def k(x0_hbm, x1_hbm, src_hbm, dst_hbm,
          so0_hbm, so1_hbm, si0_hbm, si1_hbm,
          acc, rows, gidx, sidx, sem):
        c = lax.axis_index("c")
        s = lax.axis_index("s")
        nbase = s * _SLAB

        zero16 = jnp.zeros((16,), jnp.float32)

        def _zero_rows():
            def _zrow(r, carry):
                def _zcol(j, carry2):
                    rows[r, pl.ds(j * 16, 16)] = zero16
                    return carry2
                return lax.fori_loop(0, _DA // 16, _zcol, carry)
            lax.fori_loop(0, _C, _zrow, 0)

        def _direction(g_hbm, s_hbm, x_hbm, out_hbm):
            _zero_rows()
            for kk in range(_SLAB // _C):
                pltpu.sync_copy(rows, acc.at[pl.ds(nbase + kk * _C, _C)])
            rem = _SLAB % _C
            pltpu.sync_copy(rows.at[pl.ds(0, rem)],
                            acc.at[pl.ds(nbase + (_SLAB // _C) * _C, rem)])
            plsc.subcore_barrier()

            def _chunk(kk, carry):
                j = kk * _NTILES + s
                @pl.when(j < _NCHUNKS)
                def _():
                    off = j * _C
                    pltpu.sync_copy(g_hbm.at[pl.ds(off, _C)], gidx)
                    pltpu.sync_copy(s_hbm.at[pl.ds(off, _C)], sidx)
                    pltpu.async_copy(x_hbm.at[gidx], rows, sem).wait()
                    pltpu.sync_copy(rows, acc.at[sidx], add=True)
                return carry

            lax.fori_loop(0, _CPT, _chunk, 0)
            plsc.subcore_barrier()

            @pl.when(s < _NTILES - 1)
            def _():
                pltpu.sync_copy(acc.at[pl.ds(nbase, _SLAB)],
                                out_hbm.at[pl.ds(nbase, _SLAB)])

            @pl.when(s == _NTILES - 1)
            def _():
                pltpu.sync_copy(acc.at[pl.ds(nbase, _LAST)],
                                out_hbm.at[pl.ds(nbase, _LAST)])

        @pl.when(c == 0)
        def _():
            _direction(src_hbm, dst_hbm, x0_hbm, so0_hbm)
            plsc.subcore_barrier()
            _direction(dst_hbm, src_hbm, x0_hbm, si0_hbm)

        @pl.when(c == 1)
        def _():
            _direction(src_hbm, dst_hbm, x1_hbm, so1_hbm)
            plsc.subcore_barrier()
            _direction(dst_hbm, src_hbm, x1_hbm, si1_hbm)

    return k(x0, x1, src, dst)


_BLK = 2000


def _tc_body(x_ref, so0_ref, so1_ref, si0_ref, si1_ref, rf_ref,
             wo_ref, bo_ref, wi_ref, bi_ref, ws_ref, bs_ref, wr_ref, br_ref,
             out_ref, rout_ref):
    f32 = jnp.float32
    dn_t = (((1,), (1,)), ((), ()))
    dn_n = (((1,), (0,)), ((), ()))

    wo = wo_ref[...]
    wi = wi_ref[...]
    ws = ws_ref[...]

    acc = lax.dot_general(x_ref[...], ws, dn_t,
                          preferred_element_type=f32, precision=_HIGH)
    acc += lax.dot_general(so0_ref[:, :_DH], wo[:, :_DH], dn_t,
                           preferred_element_type=f32, precision=_HIGH)
    acc += lax.dot_general(so1_ref[:, :_DH], wo[:, _DH:], dn_t,
                           preferred_element_type=f32, precision=_HIGH)
    acc += lax.dot_general(si0_ref[:, :_DH], wi[:, :_DH], dn_t,
                           preferred_element_type=f32, precision=_HIGH)
    acc += lax.dot_general(si1_ref[:, :_DH], wi[:, _DH:], dn_t,
                           preferred_element_type=f32, precision=_HIGH)

    rf = rf_ref[...]
    r_wo = lax.dot_general(rf, wo, dn_t, preferred_element_type=f32,
                           precision=_HIGH)
    r_wi = lax.dot_general(rf, wi, dn_t, preferred_element_type=f32,
                           precision=_HIGH)
    r_ws = lax.dot_general(rf, ws, dn_t, preferred_element_type=f32,
                           precision=_HIGH)
    c_o = bo_ref[...] - r_wo[0:1, :]
    c_i = bi_ref[...] - r_wi[1:2, :]
    c_s = bs_ref[...] - r_ws[2:3, :]

    acc += lax.dot_general(so0_ref[:, _DH:], jnp.broadcast_to(c_o / 16.0, (16, _D)),
                           dn_n, preferred_element_type=f32, precision=_HIGH)
    acc += lax.dot_general(si0_ref[:, _DH:], jnp.broadcast_to(c_i / 16.0, (16, _D)),
                           dn_n, preferred_element_type=f32, precision=_HIGH)
    acc += jnp.broadcast_to(c_s, acc.shape)
    out_ref[...] = acc

    @pl.when(pl.program_id(0) == 0)
    def _():
        r_wr = lax.dot_general(rf, wr_ref[...], dn_t,
                               preferred_element_type=f32, precision=_HIGH)
        rout_ref[...] = r_wr + br_ref[...]


def _tc_combine(x, so0, so1, si0, si1, rf8, W_O, b_O, W_I, b_I, W_S, b_S,
                W_R, b_R):
    rows = lambda i: (i, 0)
    full = lambda i: (0, 0)
    grid = (_N // _BLK,)
    in_specs = [
        pl.BlockSpec((_BLK, _D), rows),
        pl.BlockSpec((_BLK, _DA), rows),
        pl.BlockSpec((_BLK, _DA), rows),
        pl.BlockSpec((_BLK, _DA), rows),
        pl.BlockSpec((_BLK, _DA), rows),
        pl.BlockSpec((8, _D), full),
        pl.BlockSpec((_D, _D), full),
        pl.BlockSpec((1, _D), full),
        pl.BlockSpec((_D, _D), full),
        pl.BlockSpec((1, _D), full),
        pl.BlockSpec((_D, _D), full),
        pl.BlockSpec((1, _D), full),
        pl.BlockSpec((_D, _D), full),
        pl.BlockSpec((1, _D), full),
    ]
    out_specs = (pl.BlockSpec((_BLK, _D), rows), pl.BlockSpec((8, _D), full))
    out_shape = (jax.ShapeDtypeStruct((_N, _D), jnp.float32),
                 jax.ShapeDtypeStruct((8, _D), jnp.float32))
    return pl.pallas_call(
        _tc_body, grid=grid, in_specs=in_specs, out_specs=out_specs,
        out_shape=out_shape,
    )(x, so0, so1, si0, si1, rf8, W_O, b_O, W_I, b_I, W_S, b_S, W_R, b_R)


def kernel(x, edge_index, r_feats, W_O, b_O, W_I, b_I, W_S, b_S, W_R, b_R):
    ones = jnp.ones((_N, 16), jnp.float32)
    x0 = jnp.concatenate([x[:, :_DH], ones], axis=1)
    x1 = jnp.concatenate([x[:, _DH:], ones], axis=1)
    src = edge_index[0]
    dst = edge_index[1]

    so0, so1, si0, si1 = _sc_segment_sums(x0, x1, src, dst)

    rf8 = jnp.pad(r_feats, ((0, 5), (0, 0)))
    n_out, r8 = _tc_combine(
        x, so0, so1, si0, si1, rf8,
        W_O, b_O.reshape(1, _D), W_I, b_I.reshape(1, _D),
        W_S, b_S.reshape(1, _D), W_R, b_R.reshape(1, _D))
    return (n_out, r8[:3])


# speedup vs baseline: 4.4847x; 1.6111x over previous
import functools

import jax
import jax.numpy as jnp
from jax import lax
from jax.experimental import pallas as pl
from jax.experimental.pallas import tpu as pltpu
from jax.experimental.pallas import tpu_sc as plsc

_N = 10000
_E = 160000
_D = 256
_DH = 128
_DA = _DH + 32
_C = 128
_NTILES = 16
_CPT = 79
_EPAD = _NTILES * _CPT * _C
_SLAB = 632
_NPAD = _NTILES * _SLAB
_LAST = _N - 15 * _SLAB
_GARBAGE = _N + 64


def _sc_segment_sums(x0, x1, srcq, dstq):
    mesh = plsc.VectorSubcoreMesh(core_axis_name="c", subcore_axis_name="s")
    out = jax.ShapeDtypeStruct((_N, _DA), jnp.bfloat16)

    @functools.partial(
        pl.kernel,
        mesh=mesh,
        out_type=(out, out, out, out),
        compiler_params=pltpu.CompilerParams(use_tc_tiling_on_sc=False),
        scratch_types=[
            pltpu.VMEM_SHARED((_NPAD, _DA), jnp.bfloat16),
            pltpu.VMEM((_C, _DA), jnp.bfloat16),
            pltpu.VMEM((_C, _DA), jnp.bfloat16),
            pltpu.VMEM((_CPT, _C), jnp.int32),
            pltpu.VMEM((_CPT, _C), jnp.int32),
            pltpu.SemaphoreType.DMA,
            pltpu.SemaphoreType.DMA,
        ],
    )
    def k(x0_hbm, x1_hbm, srcq_hbm, dstq_hbm,
          so0_hbm, so1_hbm, si0_hbm, si1_hbm,
          acc, buf0, buf1, srcb, dstb, sem0, sem1):
        c = lax.axis_index("c")
        s = lax.axis_index("s")
        nbase = s * _SLAB

        pltpu.sync_copy(srcq_hbm.at[pl.ds(s * _CPT, _CPT)], srcb)
        pltpu.sync_copy(dstq_hbm.at[pl.ds(s * _CPT, _CPT)], dstb)

        zero32 = jnp.zeros((32,), jnp.bfloat16)

        def _zero_buf0():
            def _zrow(r, carry):
                def _zcol(j, carry2):
                    buf0[r, pl.ds(j * 32, 32)] = zero32
                    return carry2
                return lax.fori_loop(0, _DA // 32, _zcol, carry)
            lax.fori_loop(0, _C, _zrow, 0)

        def _direction(x_hbm, gi, si, out_hbm):
            _zero_buf0()
            for kk in range(_SLAB // _C):
                pltpu.sync_copy(buf0, acc.at[pl.ds(nbase + kk * _C, _C)])
            rem = _SLAB % _C
            pltpu.sync_copy(buf0.at[pl.ds(0, rem)],
                            acc.at[pl.ds(nbase + (_SLAB // _C) * _C, rem)])
            plsc.subcore_barrier()

            pltpu.async_copy(x_hbm.at[gi.at[0]], buf0, sem0)

            def _pair(p, carry):
                e0 = 2 * p
                pltpu.async_copy(x_hbm.at[gi.at[e0 + 1]], buf1, sem1)
                pltpu.make_async_copy(x_hbm.at[gi.at[e0]], buf0, sem0).wait()
                pltpu.sync_copy(buf0, acc.at[si.at[e0]], add=True)
                pltpu.async_copy(x_hbm.at[gi.at[e0 + 2]], buf0, sem0)
                pltpu.make_async_copy(x_hbm.at[gi.at[e0 + 1]], buf1, sem1).wait()
                pltpu.sync_copy(buf1, acc.at[si.at[e0 + 1]], add=True)
                return carry

            lax.fori_loop(0, (_CPT - 1) // 2, _pair, 0)
            pltpu.make_async_copy(x_hbm.at[gi.at[_CPT - 1]], buf0, sem0).wait()
            pltpu.sync_copy(buf0, acc.at[si.at[_CPT - 1]], add=True)
            plsc.subcore_barrier()

            @pl.when(s < _NTILES - 1)
            def _():
                pltpu.sync_copy(acc.at[pl.ds(nbase, _SLAB)],
                                out_hbm.at[pl.ds(nbase, _SLAB)])

            @pl.when(s == _NTILES - 1)
            def _():
                pltpu.sync_copy(acc.at[pl.ds(nbase, _LAST)],
                                out_hbm.at[pl.ds(nbase, _LAST)])

        @pl.when(c == 0)
        def _():
            _direction(x0_hbm, srcb, dstb, so0_hbm)
            plsc.subcore_barrier()
            _direction(x0_hbm, dstb, srcb, si0_hbm)

        @pl.when(c == 1)
        def _():
            _direction(x1_hbm, srcb, dstb, so1_hbm)
            plsc.subcore_barrier()
            _direction(x1_hbm, dstb, srcb, si1_hbm)

    return k(x0, x1, srcq, dstq)


_BLK = 2000


def _tc_body(x_ref, so0_ref, so1_ref, si0_ref, si1_ref, rf_ref,
             wo_ref, bo_ref, wi_ref, bi_ref, ws_ref, bs_ref, wr_ref, br_ref,
             out_ref, rout_ref):
    f32 = jnp.float32
    bf16 = jnp.bfloat16
    dn_t = (((1,), (1,)), ((), ()))

    wo = wo_ref[...]
    wi = wi_ref[...]
    ws = ws_ref[...]

    acc = lax.dot_general(x_ref[...], ws, dn_t, preferred_element_type=f32)
    wo_b = wo.astype(bf16)
    wi_b = wi.astype(bf16)
    acc += lax.dot_general(so0_ref[:, :_DH], wo_b[:, :_DH], dn_t,
                           preferred_element_type=f32)
    acc += lax.dot_general(so1_ref[:, :_DH], wo_b[:, _DH:], dn_t,
                           preferred_element_type=f32)
    acc += lax.dot_general(si0_ref[:, :_DH], wi_b[:, :_DH], dn_t,
                           preferred_element_type=f32)
    acc += lax.dot_general(si1_ref[:, :_DH], wi_b[:, _DH:], dn_t,
                           preferred_element_type=f32)

    rf = rf_ref[...]
    r_wo = lax.dot_general(rf, wo, dn_t, preferred_element_type=f32)
    r_wi = lax.dot_general(rf, wi, dn_t, preferred_element_type=f32)
    r_ws = lax.dot_general(rf, ws, dn_t, preferred_element_type=f32)
    c_o = bo_ref[...] - r_wo[0:1, :]
    c_i = bi_ref[...] - r_wi[1:2, :]
    c_s = bs_ref[...] - r_ws[2:3, :]

    deg_o = so0_ref[:, _DH:_DH + 1].astype(f32)
    deg_i = si0_ref[:, _DH:_DH + 1].astype(f32)
    acc += deg_o * jnp.broadcast_to(c_o, (_BLK, _D))
    acc += deg_i * jnp.broadcast_to(c_i, (_BLK, _D))
    acc += jnp.broadcast_to(c_s, acc.shape)
    out_ref[...] = acc

    @pl.when(pl.program_id(0) == 0)
    def _():
        r_wr = lax.dot_general(rf, wr_ref[...], dn_t,
                               preferred_element_type=f32)
        rout_ref[...] = r_wr + br_ref[...]


def _tc_combine(x, so0, so1, si0, si1, rf8, W_O, b_O, W_I, b_I, W_S, b_S,
                W_R, b_R):
    rows = lambda i: (i, 0)
    full = lambda i: (0, 0)
    grid = (_N // _BLK,)
    in_specs = [
        pl.BlockSpec((_BLK, _D), rows),
        pl.BlockSpec((_BLK, _DA), rows),
        pl.BlockSpec((_BLK, _DA), rows),
        pl.BlockSpec((_BLK, _DA), rows),
        pl.BlockSpec((_BLK, _DA), rows),
        pl.BlockSpec((8, _D), full),
        pl.BlockSpec((_D, _D), full),
        pl.BlockSpec((1, _D), full),
        pl.BlockSpec((_D, _D), full),
        pl.BlockSpec((1, _D), full),
        pl.BlockSpec((_D, _D), full),
        pl.BlockSpec((1, _D), full),
        pl.BlockSpec((_D, _D), full),
        pl.BlockSpec((1, _D), full),
    ]
    out_specs = (pl.BlockSpec((_BLK, _D), rows), pl.BlockSpec((8, _D), full))
    out_shape = (jax.ShapeDtypeStruct((_N, _D), jnp.float32),
                 jax.ShapeDtypeStruct((8, _D), jnp.float32))
    return pl.pallas_call(
        _tc_body, grid=grid, in_specs=in_specs, out_specs=out_specs,
        out_shape=out_shape,
    )(x, so0, so1, si0, si1, rf8, W_O, b_O, W_I, b_I, W_S, b_S, W_R, b_R)


def kernel(x, edge_index, r_feats, W_O, b_O, W_I, b_I, W_S, b_S, W_R, b_R):
    xb = x.astype(jnp.bfloat16)
    ones = jnp.ones((_N, _DA - _DH), jnp.bfloat16)
    rpad = ((0, _NPAD - _N), (0, 0))
    x0 = jnp.pad(jnp.concatenate([xb[:, :_DH], ones], axis=1), rpad)
    x1 = jnp.pad(jnp.concatenate([xb[:, _DH:], ones], axis=1), rpad)
    epad = jnp.full((_EPAD - _E,), _GARBAGE, jnp.int32)
    srcq = jnp.concatenate([edge_index[0], epad]).reshape(_EPAD // _C, _C)
    dstq = jnp.concatenate([edge_index[1], epad]).reshape(_EPAD // _C, _C)

    so0, so1, si0, si1 = _sc_segment_sums(x0, x1, srcq, dstq)

    rf8 = jnp.pad(r_feats, ((0, 5), (0, 0)))
    n_out, r8 = _tc_combine(
        x, so0, so1, si0, si1, rf8,
        W_O, b_O.reshape(1, _D), W_I, b_I.reshape(1, _D),
        W_S, b_S.reshape(1, _D), W_R, b_R.reshape(1, _D))
    return (n_out, r8[:3])


# speedup vs baseline: 5.8554x; 1.3057x over previous
import functools

import jax
import jax.numpy as jnp
from jax import lax
from jax.experimental import pallas as pl
from jax.experimental.pallas import tpu as pltpu
from jax.experimental.pallas import tpu_sc as plsc

_N = 10000
_E = 160000
_D = 256
_DH = 128
_DG = 32
_C = 128
_NTILES = 16
_CPT = 79
_EPAD = _NTILES * _CPT * _C
_SLAB = 632
_NPAD = _NTILES * _SLAB
_LAST = _N - 15 * _SLAB
_GARBAGE = _N + 64
_ZD = 158


def _sc_segment_sums(x0, x1, srcq, dstq):
    mesh = plsc.VectorSubcoreMesh(core_axis_name="c", subcore_axis_name="s")
    outf = jax.ShapeDtypeStruct((_N, _DH), jnp.bfloat16)
    outd = jax.ShapeDtypeStruct((_N, _DG), jnp.bfloat16)

    @functools.partial(
        pl.kernel,
        mesh=mesh,
        out_type=(outf, outf, outf, outf, outd, outd),
        compiler_params=pltpu.CompilerParams(use_tc_tiling_on_sc=False),
        scratch_types=[
            pltpu.VMEM_SHARED((_NPAD, _DH), jnp.bfloat16),
            pltpu.VMEM_SHARED((_NPAD, _DG), jnp.bfloat16),
            pltpu.VMEM((_C, _DH), jnp.bfloat16),
            pltpu.VMEM((_C, _DH), jnp.bfloat16),
            pltpu.VMEM((_C, _DG), jnp.bfloat16),
            pltpu.VMEM((_ZD, _DG), jnp.bfloat16),
            pltpu.VMEM((_CPT, _C), jnp.int32),
            pltpu.VMEM((_CPT, _C), jnp.int32),
            pltpu.SemaphoreType.DMA,
            pltpu.SemaphoreType.DMA,
        ],
    )
    def k(x0_hbm, x1_hbm, srcq_hbm, dstq_hbm,
          so0_hbm, so1_hbm, si0_hbm, si1_hbm, dego_hbm, degi_hbm,
          acc, accd, buf0, buf1, onesb, zbufd, srcb, dstb, sem0, sem1):
        c = lax.axis_index("c")
        s = lax.axis_index("s")
        nbase = s * _SLAB

        pltpu.sync_copy(srcq_hbm.at[pl.ds(s * _CPT, _CPT)], srcb)
        pltpu.sync_copy(dstq_hbm.at[pl.ds(s * _CPT, _CPT)], dstb)

        zero32 = jnp.zeros((_DG,), jnp.bfloat16)
        one32 = jnp.ones((_DG,), jnp.bfloat16)

        def _fill(ref, nrows, ncols, val):
            def _frow(r, carry):
                def _fcol(j, carry2):
                    ref[r, pl.ds(j * _DG, _DG)] = val
                    return carry2
                return lax.fori_loop(0, ncols // _DG, _fcol, carry)
            lax.fori_loop(0, nrows, _frow, 0)

        _fill(onesb, _C, _DG, one32)
        _fill(zbufd, _ZD, _DG, zero32)

        def _direction(x_hbm, gi, si, out_hbm, deg_hbm):
            _fill(buf0, _C, _DH, zero32)
            for kk in range(_SLAB // _C):
                pltpu.sync_copy(buf0, acc.at[pl.ds(nbase + kk * _C, _C)])
            rem = _SLAB % _C
            pltpu.sync_copy(buf0.at[pl.ds(0, rem)],
                            acc.at[pl.ds(nbase + (_SLAB // _C) * _C, rem)])
            do_deg = deg_hbm is not None
            if do_deg:
                for kk in range(_SLAB // _ZD):
                    pltpu.sync_copy(zbufd,
                                    accd.at[pl.ds(nbase + kk * _ZD, _ZD)])
            plsc.subcore_barrier()

            pltpu.async_copy(x_hbm.at[gi.at[0]], buf0, sem0)

            def _pair(p, carry):
                e0 = 2 * p
                pltpu.async_copy(x_hbm.at[gi.at[e0 + 1]], buf1, sem1)
                pltpu.make_async_copy(x_hbm.at[gi.at[e0]], buf0, sem0).wait()
                pltpu.sync_copy(buf0, acc.at[si.at[e0]], add=True)
                if do_deg:
                    pltpu.sync_copy(onesb, accd.at[si.at[e0]], add=True)
                pltpu.async_copy(x_hbm.at[gi.at[e0 + 2]], buf0, sem0)
                pltpu.make_async_copy(x_hbm.at[gi.at[e0 + 1]], buf1, sem1).wait()
                pltpu.sync_copy(buf1, acc.at[si.at[e0 + 1]], add=True)
                if do_deg:
                    pltpu.sync_copy(onesb, accd.at[si.at[e0 + 1]], add=True)
                return carry

            lax.fori_loop(0, (_CPT - 1) // 2, _pair, 0)
            pltpu.make_async_copy(x_hbm.at[gi.at[_CPT - 1]], buf0, sem0).wait()
            pltpu.sync_copy(buf0, acc.at[si.at[_CPT - 1]], add=True)
            if do_deg:
                pltpu.sync_copy(onesb, accd.at[si.at[_CPT - 1]], add=True)
            plsc.subcore_barrier()

            @pl.when(s < _NTILES - 1)
            def _():
                pltpu.sync_copy(acc.at[pl.ds(nbase, _SLAB)],
                                out_hbm.at[pl.ds(nbase, _SLAB)])
                if do_deg:
                    pltpu.sync_copy(accd.at[pl.ds(nbase, _SLAB)],
                                    deg_hbm.at[pl.ds(nbase, _SLAB)])

            @pl.when(s == _NTILES - 1)
            def _():
                pltpu.sync_copy(acc.at[pl.ds(nbase, _LAST)],
                                out_hbm.at[pl.ds(nbase, _LAST)])
                if do_deg:
                    pltpu.sync_copy(accd.at[pl.ds(nbase, _LAST)],
                                    deg_hbm.at[pl.ds(nbase, _LAST)])

        @pl.when(c == 0)
        def _():
            _direction(x0_hbm, srcb, dstb, so0_hbm, dego_hbm)
            plsc.subcore_barrier()
            _direction(x0_hbm, dstb, srcb, si0_hbm, None)

        @pl.when(c == 1)
        def _():
            _direction(x1_hbm, srcb, dstb, so1_hbm, None)
            plsc.subcore_barrier()
            _direction(x1_hbm, dstb, srcb, si1_hbm, degi_hbm)

    return k(x0, x1, srcq, dstq)


_BLK = 2000


def _tc_body(x_ref, so0_ref, so1_ref, si0_ref, si1_ref, dego_ref, degi_ref,
             rf_ref, wo_ref, bo_ref, wi_ref, bi_ref, ws_ref, bs_ref,
             wr_ref, br_ref, out_ref, rout_ref):
    f32 = jnp.float32
    bf16 = jnp.bfloat16
    dn_t = (((1,), (1,)), ((), ()))

    wo = wo_ref[...]
    wi = wi_ref[...]
    ws = ws_ref[...]

    acc = lax.dot_general(x_ref[...], ws.astype(bf16), dn_t,
                          preferred_element_type=f32)
    wo_b = wo.astype(bf16)
    wi_b = wi.astype(bf16)
    acc += lax.dot_general(so0_ref[...], wo_b[:, :_DH], dn_t,
                           preferred_element_type=f32)
    acc += lax.dot_general(so1_ref[...], wo_b[:, _DH:], dn_t,
                           preferred_element_type=f32)
    acc += lax.dot_general(si0_ref[...], wi_b[:, :_DH], dn_t,
                           preferred_element_type=f32)
    acc += lax.dot_general(si1_ref[...], wi_b[:, _DH:], dn_t,
                           preferred_element_type=f32)

    rf = rf_ref[...]
    r_wo = lax.dot_general(rf, wo, dn_t, preferred_element_type=f32)
    r_wi = lax.dot_general(rf, wi, dn_t, preferred_element_type=f32)
    r_ws = lax.dot_general(rf, ws, dn_t, preferred_element_type=f32)
    c_o = bo_ref[...] - r_wo[0:1, :]
    c_i = bi_ref[...] - r_wi[1:2, :]
    c_s = bs_ref[...] - r_ws[2:3, :]

    deg_o = dego_ref[:, 0:1].astype(f32)
    deg_i = degi_ref[:, 0:1].astype(f32)
    acc += deg_o * jnp.broadcast_to(c_o, (_BLK, _D))
    acc += deg_i * jnp.broadcast_to(c_i, (_BLK, _D))
    acc += jnp.broadcast_to(c_s, acc.shape)
    out_ref[...] = acc

    @pl.when(pl.program_id(0) == 0)
    def _():
        r_wr = lax.dot_general(rf, wr_ref[...], dn_t,
                               preferred_element_type=f32)
        rout_ref[...] = r_wr + br_ref[...]


def _tc_combine(xb, so0, so1, si0, si1, dego, degi, rf8,
                W_O, b_O, W_I, b_I, W_S, b_S, W_R, b_R):
    rows = lambda i: (i, 0)
    full = lambda i: (0, 0)
    grid = (_N // _BLK,)
    in_specs = [
        pl.BlockSpec((_BLK, _D), rows),
        pl.BlockSpec((_BLK, _DH), rows),
        pl.BlockSpec((_BLK, _DH), rows),
        pl.BlockSpec((_BLK, _DH), rows),
        pl.BlockSpec((_BLK, _DH), rows),
        pl.BlockSpec((_BLK, _DG), rows),
        pl.BlockSpec((_BLK, _DG), rows),
        pl.BlockSpec((8, _D), full),
        pl.BlockSpec((_D, _D), full),
        pl.BlockSpec((1, _D), full),
        pl.BlockSpec((_D, _D), full),
        pl.BlockSpec((1, _D), full),
        pl.BlockSpec((_D, _D), full),
        pl.BlockSpec((1, _D), full),
        pl.BlockSpec((_D, _D), full),
        pl.BlockSpec((1, _D), full),
    ]
    out_specs = (pl.BlockSpec((_BLK, _D), rows), pl.BlockSpec((8, _D), full))
    out_shape = (jax.ShapeDtypeStruct((_N, _D), jnp.float32),
                 jax.ShapeDtypeStruct((8, _D), jnp.float32))
    return pl.pallas_call(
        _tc_body, grid=grid, in_specs=in_specs, out_specs=out_specs,
        out_shape=out_shape,
    )(xb, so0, so1, si0, si1, dego, degi, rf8,
      W_O, b_O, W_I, b_I, W_S, b_S, W_R, b_R)


def kernel(x, edge_index, r_feats, W_O, b_O, W_I, b_I, W_S, b_S, W_R, b_R):
    xb = x.astype(jnp.bfloat16)
    rpad = ((0, _NPAD - _N), (0, 0))
    x0 = jnp.pad(xb[:, :_DH], rpad)
    x1 = jnp.pad(xb[:, _DH:], rpad)
    epad = jnp.full((_EPAD - _E,), _GARBAGE, jnp.int32)
    srcq = jnp.concatenate([edge_index[0], epad]).reshape(_EPAD // _C, _C)
    dstq = jnp.concatenate([edge_index[1], epad]).reshape(_EPAD // _C, _C)

    so0, so1, si0, si1, dego, degi = _sc_segment_sums(x0, x1, srcq, dstq)

    rf8 = jnp.pad(r_feats, ((0, 5), (0, 0)))
    n_out, r8 = _tc_combine(
        xb, so0, so1, si0, si1, dego, degi, rf8,
        W_O, b_O.reshape(1, _D), W_I, b_I.reshape(1, _D),
        W_S, b_S.reshape(1, _D), W_R, b_R.reshape(1, _D))
    return (n_out, r8[:3])


# speedup vs baseline: 6.8488x; 1.1697x over previous
import functools

import jax
import jax.numpy as jnp
from jax import lax
from jax.experimental import pallas as pl
from jax.experimental.pallas import tpu as pltpu
from jax.experimental.pallas import tpu_sc as plsc

_N = 10000
_E = 160000
_D = 256
_DH = 128
_DG = 32
_C = 80
_NTILES = 16
_CPT = (_E // _C) // _NTILES
_SLAB = 632
_NPAD = _NTILES * _SLAB
_LAST = _N - 15 * _SLAB
_ZD = 158


def _sc_segment_sums(x0, x1, srcq, dstq):
    mesh = plsc.VectorSubcoreMesh(core_axis_name="c", subcore_axis_name="s")

    @functools.partial(
        pl.kernel,
        mesh=mesh,
        out_type=(jax.ShapeDtypeStruct((4, _N, _DH), jnp.bfloat16),
                  jax.ShapeDtypeStruct((2, _N, _DG), jnp.bfloat16)),
        compiler_params=pltpu.CompilerParams(use_tc_tiling_on_sc=False),
        scratch_types=[
            pltpu.VMEM_SHARED((_NPAD, _DH), jnp.bfloat16),
            pltpu.VMEM_SHARED((_NPAD, _DG), jnp.bfloat16),
            pltpu.VMEM((_C, _DH), jnp.bfloat16),
            pltpu.VMEM((_C, _DH), jnp.bfloat16),
            pltpu.VMEM((_C, _DG), jnp.bfloat16),
            pltpu.VMEM((_ZD, _DG), jnp.bfloat16),
            pltpu.VMEM((_CPT, _C), jnp.int32),
            pltpu.VMEM((_CPT, _C), jnp.int32),
            pltpu.SemaphoreType.DMA,
            pltpu.SemaphoreType.DMA,
        ],
    )
    def k(x0_hbm, x1_hbm, srcq_hbm, dstq_hbm, sums_hbm, degs_hbm,
          acc, accd, buf0, buf1, onesb, zbufd, srcb, dstb, sem0, sem1):
        c = lax.axis_index("c")
        s = lax.axis_index("s")
        nbase = s * _SLAB

        pltpu.sync_copy(srcq_hbm.at[pl.ds(s * _CPT, _CPT)], srcb)
        pltpu.sync_copy(dstq_hbm.at[pl.ds(s * _CPT, _CPT)], dstb)

        zero32 = jnp.zeros((_DG,), jnp.bfloat16)
        one32 = jnp.ones((_DG,), jnp.bfloat16)

        def _fill(ref, nrows, ncols, val):
            def _frow(r, carry):
                def _fcol(j, carry2):
                    ref[r, pl.ds(j * _DG, _DG)] = val
                    return carry2
                return lax.fori_loop(0, ncols // _DG, _fcol, carry)
            lax.fori_loop(0, nrows, _frow, 0)

        _fill(onesb, _C, _DG, one32)
        _fill(zbufd, _ZD, _DG, zero32)

        def _direction(x_hbm, gi, si, out_slot, deg_slot):
            _fill(buf0, _C, _DH, zero32)
            for kk in range(_SLAB // _C):
                pltpu.sync_copy(buf0, acc.at[pl.ds(nbase + kk * _C, _C)])
            rem = _SLAB % _C
            pltpu.sync_copy(buf0.at[pl.ds(0, rem)],
                            acc.at[pl.ds(nbase + (_SLAB // _C) * _C, rem)])
            do_deg = deg_slot is not None
            if do_deg:
                for kk in range(_SLAB // _ZD):
                    pltpu.sync_copy(zbufd,
                                    accd.at[pl.ds(nbase + kk * _ZD, _ZD)])
            plsc.subcore_barrier()

            pltpu.async_copy(x_hbm.at[gi.at[0]], buf0, sem0)

            def _pair(p, carry):
                e0 = 2 * p
                pltpu.async_copy(x_hbm.at[gi.at[e0 + 1]], buf1, sem1)
                pltpu.make_async_copy(x_hbm.at[gi.at[e0]], buf0, sem0).wait()
                pltpu.sync_copy(buf0, acc.at[si.at[e0]], add=True)
                if do_deg:
                    pltpu.sync_copy(onesb, accd.at[si.at[e0]], add=True)
                pltpu.async_copy(x_hbm.at[gi.at[e0 + 2]], buf0, sem0)
                pltpu.make_async_copy(x_hbm.at[gi.at[e0 + 1]], buf1, sem1).wait()
                pltpu.sync_copy(buf1, acc.at[si.at[e0 + 1]], add=True)
                if do_deg:
                    pltpu.sync_copy(onesb, accd.at[si.at[e0 + 1]], add=True)
                return carry

            lax.fori_loop(0, (_CPT - 1) // 2, _pair, 0)
            pltpu.make_async_copy(x_hbm.at[gi.at[_CPT - 1]], buf0, sem0).wait()
            pltpu.sync_copy(buf0, acc.at[si.at[_CPT - 1]], add=True)
            if do_deg:
                pltpu.sync_copy(onesb, accd.at[si.at[_CPT - 1]], add=True)
            plsc.subcore_barrier()

            @pl.when(s < _NTILES - 1)
            def _():
                pltpu.sync_copy(acc.at[pl.ds(nbase, _SLAB)],
                                sums_hbm.at[out_slot, pl.ds(nbase, _SLAB)])
                if do_deg:
                    pltpu.sync_copy(accd.at[pl.ds(nbase, _SLAB)],
                                    degs_hbm.at[deg_slot, pl.ds(nbase, _SLAB)])

            @pl.when(s == _NTILES - 1)
            def _():
                pltpu.sync_copy(acc.at[pl.ds(nbase, _LAST)],
                                sums_hbm.at[out_slot, pl.ds(nbase, _LAST)])
                if do_deg:
                    pltpu.sync_copy(accd.at[pl.ds(nbase, _LAST)],
                                    degs_hbm.at[deg_slot, pl.ds(nbase, _LAST)])

        @pl.when(c == 0)
        def _():
            _direction(x0_hbm, srcb, dstb, 0, 0)
            plsc.subcore_barrier()
            _direction(x0_hbm, dstb, srcb, 2, None)

        @pl.when(c == 1)
        def _():
            _direction(x1_hbm, srcb, dstb, 1, None)
            plsc.subcore_barrier()
            _direction(x1_hbm, dstb, srcb, 3, 1)

    return k(x0, x1, srcq, dstq)


_BLK = 2000


def _tc_body(x_ref, sums_ref, degs_ref,
             rf_ref, wo_ref, bo_ref, wi_ref, bi_ref, ws_ref, bs_ref,
             wr_ref, br_ref, out_ref, rout_ref):
    f32 = jnp.float32
    bf16 = jnp.bfloat16
    dn_t = (((1,), (1,)), ((), ()))

    wo = wo_ref[...]
    wi = wi_ref[...]
    ws = ws_ref[...]

    acc = lax.dot_general(x_ref[...], ws.astype(bf16), dn_t,
                          preferred_element_type=f32)
    wo_b = wo.astype(bf16)
    wi_b = wi.astype(bf16)
    acc += lax.dot_general(sums_ref[0], wo_b[:, :_DH], dn_t,
                           preferred_element_type=f32)
    acc += lax.dot_general(sums_ref[1], wo_b[:, _DH:], dn_t,
                           preferred_element_type=f32)
    acc += lax.dot_general(sums_ref[2], wi_b[:, :_DH], dn_t,
                           preferred_element_type=f32)
    acc += lax.dot_general(sums_ref[3], wi_b[:, _DH:], dn_t,
                           preferred_element_type=f32)

    rf = rf_ref[...]
    r_wo = lax.dot_general(rf, wo, dn_t, preferred_element_type=f32)
    r_wi = lax.dot_general(rf, wi, dn_t, preferred_element_type=f32)
    r_ws = lax.dot_general(rf, ws, dn_t, preferred_element_type=f32)
    c_o = bo_ref[...] - r_wo[0:1, :]
    c_i = bi_ref[...] - r_wi[1:2, :]
    c_s = bs_ref[...] - r_ws[2:3, :]

    deg_o = degs_ref[0, :, 0:1].astype(f32)
    deg_i = degs_ref[1, :, 0:1].astype(f32)
    acc += deg_o * jnp.broadcast_to(c_o, (_BLK, _D))
    acc += deg_i * jnp.broadcast_to(c_i, (_BLK, _D))
    acc += jnp.broadcast_to(c_s, acc.shape)
    out_ref[...] = acc

    @pl.when(pl.program_id(0) == 0)
    def _():
        r_wr = lax.dot_general(rf, wr_ref[...], dn_t,
                               preferred_element_type=f32)
        rout_ref[...] = r_wr + br_ref[...]


def _tc_combine(xb, sums, degs, rf8, W_O, b_O, W_I, b_I, W_S, b_S, W_R, b_R):
    rows = lambda i: (i, 0)
    rows3 = lambda i: (0, i, 0)
    full = lambda i: (0, 0)
    grid = (_N // _BLK,)
    in_specs = [
        pl.BlockSpec((_BLK, _D), rows),
        pl.BlockSpec((4, _BLK, _DH), rows3),
        pl.BlockSpec((2, _BLK, _DG), rows3),
        pl.BlockSpec((8, _D), full),
        pl.BlockSpec((_D, _D), full),
        pl.BlockSpec((1, _D), full),
        pl.BlockSpec((_D, _D), full),
        pl.BlockSpec((1, _D), full),
        pl.BlockSpec((_D, _D), full),
        pl.BlockSpec((1, _D), full),
        pl.BlockSpec((_D, _D), full),
        pl.BlockSpec((1, _D), full),
    ]
    out_specs = (pl.BlockSpec((_BLK, _D), rows), pl.BlockSpec((8, _D), full))
    out_shape = (jax.ShapeDtypeStruct((_N, _D), jnp.float32),
                 jax.ShapeDtypeStruct((8, _D), jnp.float32))
    return pl.pallas_call(
        _tc_body, grid=grid, in_specs=in_specs, out_specs=out_specs,
        out_shape=out_shape,
    )(xb, sums, degs, rf8, W_O, b_O, W_I, b_I, W_S, b_S, W_R, b_R)


def kernel(x, edge_index, r_feats, W_O, b_O, W_I, b_I, W_S, b_S, W_R, b_R):
    xb = x.astype(jnp.bfloat16)
    x0 = xb[:, :_DH]
    x1 = xb[:, _DH:]
    eq = edge_index.reshape(2, _E // _C, _C)
    srcq = eq[0]
    dstq = eq[1]

    sums, degs = _sc_segment_sums(x0, x1, srcq, dstq)

    rf8 = jnp.pad(r_feats, ((0, 5), (0, 0)))
    n_out, r8 = _tc_combine(
        xb, sums, degs, rf8,
        W_O, b_O.reshape(1, _D), W_I, b_I.reshape(1, _D),
        W_S, b_S.reshape(1, _D), W_R, b_R.reshape(1, _D))
    return (n_out, r8[:3])


# speedup vs baseline: 7.2052x; 1.0520x over previous
import functools

import jax
import jax.numpy as jnp
from jax import lax
from jax.experimental import pallas as pl
from jax.experimental.pallas import tpu as pltpu
from jax.experimental.pallas import tpu_sc as plsc

_N = 10000
_E = 160000
_D = 256
_DH = 128
_DG = 32
_C = 80
_NTILES = 16
_CPT = (_E // _C) // _NTILES
_SLAB = 632
_NPAD = _NTILES * _SLAB
_LAST = _N - 15 * _SLAB
_ZD = 158
_WQ = 79


def _sc_segment_sums(x0, x1, srcq, dstq):
    mesh = plsc.VectorSubcoreMesh(core_axis_name="c", subcore_axis_name="s")

    @functools.partial(
        pl.kernel,
        mesh=mesh,
        out_type=(jax.ShapeDtypeStruct((4, _N, _DH), jnp.float32),
                  jax.ShapeDtypeStruct((2, _N, _DG), jnp.bfloat16)),
        compiler_params=pltpu.CompilerParams(use_tc_tiling_on_sc=False),
        scratch_types=[
            pltpu.VMEM_SHARED((_NPAD, _DH), jnp.bfloat16),
            pltpu.VMEM_SHARED((_NPAD, _DG), jnp.bfloat16),
            pltpu.VMEM((_C, _DH), jnp.bfloat16),
            pltpu.VMEM((_C, _DH), jnp.bfloat16),
            pltpu.VMEM((_C, _DG), jnp.bfloat16),
            pltpu.VMEM((_ZD, _DG), jnp.bfloat16),
            pltpu.VMEM((_WQ, _DH), jnp.bfloat16),
            pltpu.VMEM((_WQ, _DH), jnp.float32),
            pltpu.VMEM((_CPT, _C), jnp.int32),
            pltpu.VMEM((_CPT, _C), jnp.int32),
            pltpu.SemaphoreType.DMA,
            pltpu.SemaphoreType.DMA,
        ],
    )
    def k(x0_hbm, x1_hbm, srcq_hbm, dstq_hbm, sums_hbm, degs_hbm,
          acc, accd, buf0, buf1, onesb, zbufd, wstage, wout,
          srcb, dstb, sem0, sem1):
        c = lax.axis_index("c")
        s = lax.axis_index("s")
        nbase = s * _SLAB

        pltpu.sync_copy(srcq_hbm.at[pl.ds(s * _CPT, _CPT)], srcb)
        pltpu.sync_copy(dstq_hbm.at[pl.ds(s * _CPT, _CPT)], dstb)

        zero32 = jnp.zeros((_DG,), jnp.bfloat16)
        one32 = jnp.ones((_DG,), jnp.bfloat16)

        def _fill(ref, nrows, ncols, val):
            def _frow(r, carry):
                def _fcol(j, carry2):
                    ref[r, pl.ds(j * _DG, _DG)] = val
                    return carry2
                return lax.fori_loop(0, ncols // _DG, _fcol, carry)
            lax.fori_loop(0, nrows, _frow, 0)

        _fill(onesb, _C, _DG, one32)
        _fill(zbufd, _ZD, _DG, zero32)

        def _convert_stage():
            def _cr(r, carry):
                def _cg(j, carry2):
                    v = wstage[r, pl.ds(j * 32, 32)].astype(jnp.float32)
                    wout[r, pl.ds(j * 32, 16)] = v[0:16]
                    wout[r, pl.ds(j * 32 + 16, 16)] = v[16:32]
                    return carry2
                return lax.fori_loop(0, _DH // 32, _cg, carry)
            lax.fori_loop(0, _WQ, _cr, 0)

        def _direction(x_hbm, gi, si, out_slot, deg_slot):
            _fill(buf0, _C, _DH, zero32)
            for kk in range(_SLAB // _C):
                pltpu.sync_copy(buf0, acc.at[pl.ds(nbase + kk * _C, _C)])
            rem = _SLAB % _C
            pltpu.sync_copy(buf0.at[pl.ds(0, rem)],
                            acc.at[pl.ds(nbase + (_SLAB // _C) * _C, rem)])
            do_deg = deg_slot is not None
            if do_deg:
                for kk in range(_SLAB // _ZD):
                    pltpu.sync_copy(zbufd,
                                    accd.at[pl.ds(nbase + kk * _ZD, _ZD)])
            plsc.subcore_barrier()

            pltpu.async_copy(x_hbm.at[gi.at[0]], buf0, sem0)

            def _pair(p, carry):
                e0 = 2 * p
                pltpu.async_copy(x_hbm.at[gi.at[e0 + 1]], buf1, sem1)
                pltpu.make_async_copy(x_hbm.at[gi.at[e0]], buf0, sem0).wait()
                pltpu.sync_copy(buf0, acc.at[si.at[e0]], add=True)
                if do_deg:
                    pltpu.sync_copy(onesb, accd.at[si.at[e0]], add=True)
                pltpu.async_copy(x_hbm.at[gi.at[e0 + 2]], buf0, sem0)
                pltpu.make_async_copy(x_hbm.at[gi.at[e0 + 1]], buf1, sem1).wait()
                pltpu.sync_copy(buf1, acc.at[si.at[e0 + 1]], add=True)
                if do_deg:
                    pltpu.sync_copy(onesb, accd.at[si.at[e0 + 1]], add=True)
                return carry

            lax.fori_loop(0, (_CPT - 1) // 2, _pair, 0)
            pltpu.make_async_copy(x_hbm.at[gi.at[_CPT - 1]], buf0, sem0).wait()
            pltpu.sync_copy(buf0, acc.at[si.at[_CPT - 1]], add=True)
            if do_deg:
                pltpu.sync_copy(onesb, accd.at[si.at[_CPT - 1]], add=True)
            plsc.subcore_barrier()

            def _wb_chunk(q, nrows):
                pltpu.sync_copy(acc.at[pl.ds(nbase + q * _WQ, _WQ)], wstage)
                _convert_stage()
                pltpu.sync_copy(
                    wout.at[pl.ds(0, nrows)],
                    sums_hbm.at[out_slot, pl.ds(nbase + q * _WQ, nrows)])

            @pl.when(s < _NTILES - 1)
            def _():
                def _q(q, carry):
                    _wb_chunk(q, _WQ)
                    return carry
                lax.fori_loop(0, _SLAB // _WQ, _q, 0)
                if do_deg:
                    pltpu.sync_copy(accd.at[pl.ds(nbase, _SLAB)],
                                    degs_hbm.at[deg_slot, pl.ds(nbase, _SLAB)])

            @pl.when(s == _NTILES - 1)
            def _():
                def _q(q, carry):
                    _wb_chunk(q, _WQ)
                    return carry
                lax.fori_loop(0, _LAST // _WQ, _q, 0)
                _wb_chunk(_LAST // _WQ, _LAST % _WQ)
                if do_deg:
                    pltpu.sync_copy(accd.at[pl.ds(nbase, _LAST)],
                                    degs_hbm.at[deg_slot, pl.ds(nbase, _LAST)])

        @pl.when(c == 0)
        def _():
            _direction(x0_hbm, srcb, dstb, 0, 0)
            plsc.subcore_barrier()
            _direction(x0_hbm, dstb, srcb, 2, None)

        @pl.when(c == 1)
        def _():
            _direction(x1_hbm, srcb, dstb, 1, None)
            plsc.subcore_barrier()
            _direction(x1_hbm, dstb, srcb, 3, 1)

    return k(x0, x1, srcq, dstq)


_BLK = 2000


def _tc_body(x_ref, sums_ref, degs_ref, weo_ref,
             rf_ref, wo_ref, bo_ref, wi_ref, bi_ref, ws_ref, bs_ref,
             wr_ref, br_ref, out_ref, rout_ref):
    f32 = jnp.float32
    bf16 = jnp.bfloat16
    dn_t = (((1,), (1,)), ((), ()))

    wo = wo_ref[...]
    wi = wi_ref[...]
    ws = ws_ref[...]

    acc = lax.dot_general(x_ref[...], ws.astype(bf16), dn_t,
                          preferred_element_type=f32)
    for d in range(4):
        acc += lax.dot_general(sums_ref[d].astype(bf16),
                               weo_ref[d].astype(bf16), dn_t,
                               preferred_element_type=f32)

    rf = rf_ref[...]
    r_wo = lax.dot_general(rf, wo, dn_t, preferred_element_type=f32)
    r_wi = lax.dot_general(rf, wi, dn_t, preferred_element_type=f32)
    r_ws = lax.dot_general(rf, ws, dn_t, preferred_element_type=f32)
    c_o = bo_ref[...] - r_wo[0:1, :]
    c_i = bi_ref[...] - r_wi[1:2, :]
    c_s = bs_ref[...] - r_ws[2:3, :]

    deg_o = degs_ref[0, :, 0:1].astype(f32)
    deg_i = degs_ref[1, :, 0:1].astype(f32)
    acc += deg_o * jnp.broadcast_to(c_o, (_BLK, _D))
    acc += deg_i * jnp.broadcast_to(c_i, (_BLK, _D))
    acc += jnp.broadcast_to(c_s, acc.shape)
    out_ref[...] = acc

    @pl.when(pl.program_id(0) == 0)
    def _():
        r_wr = lax.dot_general(rf, wr_ref[...], dn_t,
                               preferred_element_type=f32)
        rout_ref[...] = r_wr + br_ref[...]


def _tc_combine(xb, sums, degs, weo, rf8,
                W_O, b_O, W_I, b_I, W_S, b_S, W_R, b_R):
    rows = lambda i: (i, 0)
    rows3 = lambda i: (0, i, 0)
    full = lambda i: (0, 0)
    full3 = lambda i: (0, 0, 0)
    grid = (_N // _BLK,)
    in_specs = [
        pl.BlockSpec((_BLK, _D), rows),
        pl.BlockSpec((4, _BLK, _DH), rows3),
        pl.BlockSpec((2, _BLK, _DG), rows3),
        pl.BlockSpec((4, _D, _DH), full3),
        pl.BlockSpec((8, _D), full),
        pl.BlockSpec((_D, _D), full),
        pl.BlockSpec((1, _D), full),
        pl.BlockSpec((_D, _D), full),
        pl.BlockSpec((1, _D), full),
        pl.BlockSpec((_D, _D), full),
        pl.BlockSpec((1, _D), full),
        pl.BlockSpec((_D, _D), full),
        pl.BlockSpec((1, _D), full),
    ]
    out_specs = (pl.BlockSpec((_BLK, _D), rows), pl.BlockSpec((8, _D), full))
    out_shape = (jax.ShapeDtypeStruct((_N, _D), jnp.float32),
                 jax.ShapeDtypeStruct((8, _D), jnp.float32))
    return pl.pallas_call(
        _tc_body, grid=grid, in_specs=in_specs, out_specs=out_specs,
        out_shape=out_shape,
    )(xb, sums, degs, weo, rf8, W_O, b_O, W_I, b_I, W_S, b_S, W_R, b_R)


def kernel(x, edge_index, r_feats, W_O, b_O, W_I, b_I, W_S, b_S, W_R, b_R):
    xb = x.astype(jnp.bfloat16)
    x0 = xb[:, :_DH]
    x1 = xb[:, _DH:]
    eq = edge_index.reshape(2, _E // _C, _C)
    srcq = eq[0]
    dstq = eq[1]

    sums, degs = _sc_segment_sums(x0, x1, srcq, dstq)

    weo = jnp.stack([W_O[:, :_DH], W_O[:, _DH:], W_I[:, :_DH], W_I[:, _DH:]])

    rf8 = jnp.pad(r_feats, ((0, 5), (0, 0)))
    n_out, r8 = _tc_combine(
        xb, sums, degs, weo, rf8,
        W_O, b_O.reshape(1, _D), W_I, b_I.reshape(1, _D),
        W_S, b_S.reshape(1, _D), W_R, b_R.reshape(1, _D))
    return (n_out, r8[:3])


# speedup vs baseline: 7.3468x; 1.0197x over previous
import functools

import jax
import jax.numpy as jnp
from jax import lax
from jax.experimental import pallas as pl
from jax.experimental.pallas import tpu as pltpu
from jax.experimental.pallas import tpu_sc as plsc

_N = 10000
_E = 160000
_D = 256
_DH = 128
_DG = 32
_C = 80
_NTILES = 16
_CPT = (_E // _C) // _NTILES
_SLAB = 632
_NPAD = _NTILES * _SLAB
_LAST = _N - 15 * _SLAB
_ZD = 158
_WQ = 79


def _sc_segment_sums(x0, x1, srcq, dstq):
    mesh = plsc.VectorSubcoreMesh(core_axis_name="c", subcore_axis_name="s")

    @functools.partial(
        pl.kernel,
        mesh=mesh,
        out_type=(jax.ShapeDtypeStruct((4, _N, _DH), jnp.float32),
                  jax.ShapeDtypeStruct((2, _N, _DG), jnp.bfloat16)),
        compiler_params=pltpu.CompilerParams(use_tc_tiling_on_sc=False),
        scratch_types=[
            pltpu.VMEM_SHARED((_NPAD, _DH), jnp.bfloat16),
            pltpu.VMEM_SHARED((_NPAD, _DG), jnp.bfloat16),
            pltpu.VMEM((_C, _DH), jnp.bfloat16),
            pltpu.VMEM((_C, _DH), jnp.bfloat16),
            pltpu.VMEM((_C, _DG), jnp.bfloat16),
            pltpu.VMEM((_ZD, _DG), jnp.bfloat16),
            pltpu.VMEM((_WQ, _DH), jnp.bfloat16),
            pltpu.VMEM((_WQ, _DH), jnp.bfloat16),
            pltpu.VMEM((_WQ, _DH), jnp.float32),
            pltpu.VMEM((_WQ, _DH), jnp.float32),
            pltpu.VMEM((_CPT, _C), jnp.int32),
            pltpu.VMEM((_CPT, _C), jnp.int32),
            pltpu.SemaphoreType.DMA,
            pltpu.SemaphoreType.DMA,
            pltpu.SemaphoreType.DMA,
            pltpu.SemaphoreType.DMA,
        ],
    )
    def k(x0_hbm, x1_hbm, srcq_hbm, dstq_hbm, sums_hbm, degs_hbm,
          acc, accd, buf0, buf1, onesb, zbufd, wstage0, wstage1,
          wout0, wout1, srcb, dstb, sem0, sem1, semo0, semo1):
        c = lax.axis_index("c")
        s = lax.axis_index("s")
        nbase = s * _SLAB

        pltpu.sync_copy(srcq_hbm.at[pl.ds(s * _CPT, _CPT)], srcb)
        pltpu.sync_copy(dstq_hbm.at[pl.ds(s * _CPT, _CPT)], dstb)

        zero32 = jnp.zeros((_DG,), jnp.bfloat16)
        one32 = jnp.ones((_DG,), jnp.bfloat16)

        def _fill(ref, nrows, ncols, val):
            def _frow(r, carry):
                def _fcol(j, carry2):
                    ref[r, pl.ds(j * _DG, _DG)] = val
                    return carry2
                return lax.fori_loop(0, ncols // _DG, _fcol, carry)
            lax.fori_loop(0, nrows, _frow, 0)

        _fill(onesb, _C, _DG, one32)
        _fill(zbufd, _ZD, _DG, zero32)

        def _convert_stage(wstage, wout):
            def _cr(r, carry):
                def _cg(j, carry2):
                    v = wstage[r, pl.ds(j * 32, 32)].astype(jnp.float32)
                    wout[r, pl.ds(j * 32, 16)] = v[0:16]
                    wout[r, pl.ds(j * 32 + 16, 16)] = v[16:32]
                    return carry2
                return lax.fori_loop(0, _DH // 32, _cg, carry)
            lax.fori_loop(0, _WQ, _cr, 0)

        def _direction(x_hbm, gi, si, out_slot, deg_slot):
            _fill(buf0, _C, _DH, zero32)
            for kk in range(_SLAB // _C):
                pltpu.sync_copy(buf0, acc.at[pl.ds(nbase + kk * _C, _C)])
            rem = _SLAB % _C
            pltpu.sync_copy(buf0.at[pl.ds(0, rem)],
                            acc.at[pl.ds(nbase + (_SLAB // _C) * _C, rem)])
            do_deg = deg_slot is not None
            if do_deg:
                for kk in range(_SLAB // _ZD):
                    pltpu.sync_copy(zbufd,
                                    accd.at[pl.ds(nbase + kk * _ZD, _ZD)])
            plsc.subcore_barrier()

            pltpu.async_copy(x_hbm.at[gi.at[0]], buf0, sem0)

            def _pair(p, carry):
                e0 = 2 * p
                pltpu.async_copy(x_hbm.at[gi.at[e0 + 1]], buf1, sem1)
                pltpu.make_async_copy(x_hbm.at[gi.at[e0]], buf0, sem0).wait()
                pltpu.sync_copy(buf0, acc.at[si.at[e0]], add=True)
                if do_deg:
                    pltpu.sync_copy(onesb, accd.at[si.at[e0]], add=True)
                pltpu.async_copy(x_hbm.at[gi.at[e0 + 2]], buf0, sem0)
                pltpu.make_async_copy(x_hbm.at[gi.at[e0 + 1]], buf1, sem1).wait()
                pltpu.sync_copy(buf1, acc.at[si.at[e0 + 1]], add=True)
                if do_deg:
                    pltpu.sync_copy(onesb, accd.at[si.at[e0 + 1]], add=True)
                return carry

            lax.fori_loop(0, (_CPT - 1) // 2, _pair, 0)
            pltpu.make_async_copy(x_hbm.at[gi.at[_CPT - 1]], buf0, sem0).wait()
            pltpu.sync_copy(buf0, acc.at[si.at[_CPT - 1]], add=True)
            if do_deg:
                pltpu.sync_copy(onesb, accd.at[si.at[_CPT - 1]], add=True)
            plsc.subcore_barrier()

            def _writeback(chunk_rows):
                nq = len(chunk_rows)
                st = (wstage0, wstage1)
                ot = (wout0, wout1)
                si = (sem0, sem1)
                so = (semo0, semo1)

                def _in_args(q):
                    return (acc.at[pl.ds(nbase + q * _WQ, _WQ)], st[q % 2],
                            si[q % 2])

                def _out_args(q):
                    return (ot[q % 2].at[pl.ds(0, chunk_rows[q])],
                            sums_hbm.at[out_slot,
                                        pl.ds(nbase + q * _WQ, chunk_rows[q])],
                            so[q % 2])

                pltpu.async_copy(*_in_args(0))
                for q in range(nq):
                    if q + 1 < nq:
                        pltpu.async_copy(*_in_args(q + 1))
                    pltpu.make_async_copy(*_in_args(q)).wait()
                    if q >= 2:
                        pltpu.make_async_copy(*_out_args(q - 2)).wait()
                    _convert_stage(st[q % 2], ot[q % 2])
                    pltpu.async_copy(*_out_args(q))
                for q in range(max(nq - 2, 0), nq):
                    pltpu.make_async_copy(*_out_args(q)).wait()

            @pl.when(s < _NTILES - 1)
            def _():
                _writeback([_WQ] * (_SLAB // _WQ))
                if do_deg:
                    pltpu.sync_copy(accd.at[pl.ds(nbase, _SLAB)],
                                    degs_hbm.at[deg_slot, pl.ds(nbase, _SLAB)])

            @pl.when(s == _NTILES - 1)
            def _():
                _writeback([_WQ] * (_LAST // _WQ) + [_LAST % _WQ])
                if do_deg:
                    pltpu.sync_copy(accd.at[pl.ds(nbase, _LAST)],
                                    degs_hbm.at[deg_slot, pl.ds(nbase, _LAST)])

        @pl.when(c == 0)
        def _():
            _direction(x0_hbm, srcb, dstb, 0, 0)
            plsc.subcore_barrier()
            _direction(x0_hbm, dstb, srcb, 2, None)

        @pl.when(c == 1)
        def _():
            _direction(x1_hbm, srcb, dstb, 1, None)
            plsc.subcore_barrier()
            _direction(x1_hbm, dstb, srcb, 3, 1)

    return k(x0, x1, srcq, dstq)


_BLK = 2000


def _tc_body(x_ref, sums_ref, degs_ref, weo_ref,
             rf_ref, wo_ref, bo_ref, wi_ref, bi_ref, ws_ref, bs_ref,
             wr_ref, br_ref, out_ref, rout_ref):
    f32 = jnp.float32
    bf16 = jnp.bfloat16
    dn_t = (((1,), (1,)), ((), ()))

    wo = wo_ref[...]
    wi = wi_ref[...]
    ws = ws_ref[...]

    acc = lax.dot_general(x_ref[...], ws.astype(bf16), dn_t,
                          preferred_element_type=f32)
    for d in range(4):
        acc += lax.dot_general(sums_ref[d].astype(bf16),
                               weo_ref[d].astype(bf16), dn_t,
                               preferred_element_type=f32)

    rf = rf_ref[...]
    r_wo = lax.dot_general(rf, wo, dn_t, preferred_element_type=f32)
    r_wi = lax.dot_general(rf, wi, dn_t, preferred_element_type=f32)
    r_ws = lax.dot_general(rf, ws, dn_t, preferred_element_type=f32)
    c_o = bo_ref[...] - r_wo[0:1, :]
    c_i = bi_ref[...] - r_wi[1:2, :]
    c_s = bs_ref[...] - r_ws[2:3, :]

    deg_o = degs_ref[0, :, 0:1].astype(f32)
    deg_i = degs_ref[1, :, 0:1].astype(f32)
    acc += deg_o * jnp.broadcast_to(c_o, (_BLK, _D))
    acc += deg_i * jnp.broadcast_to(c_i, (_BLK, _D))
    acc += jnp.broadcast_to(c_s, acc.shape)
    out_ref[...] = acc

    @pl.when(pl.program_id(0) == 0)
    def _():
        r_wr = lax.dot_general(rf, wr_ref[...], dn_t,
                               preferred_element_type=f32)
        rout_ref[...] = r_wr + br_ref[...]


def _tc_combine(xb, sums, degs, weo, rf8,
                W_O, b_O, W_I, b_I, W_S, b_S, W_R, b_R):
    rows = lambda i: (i, 0)
    rows3 = lambda i: (0, i, 0)
    full = lambda i: (0, 0)
    full3 = lambda i: (0, 0, 0)
    grid = (_N // _BLK,)
    in_specs = [
        pl.BlockSpec((_BLK, _D), rows),
        pl.BlockSpec((4, _BLK, _DH), rows3),
        pl.BlockSpec((2, _BLK, _DG), rows3),
        pl.BlockSpec((4, _D, _DH), full3),
        pl.BlockSpec((8, _D), full),
        pl.BlockSpec((_D, _D), full),
        pl.BlockSpec((1, _D), full),
        pl.BlockSpec((_D, _D), full),
        pl.BlockSpec((1, _D), full),
        pl.BlockSpec((_D, _D), full),
        pl.BlockSpec((1, _D), full),
        pl.BlockSpec((_D, _D), full),
        pl.BlockSpec((1, _D), full),
    ]
    out_specs = (pl.BlockSpec((_BLK, _D), rows), pl.BlockSpec((8, _D), full))
    out_shape = (jax.ShapeDtypeStruct((_N, _D), jnp.float32),
                 jax.ShapeDtypeStruct((8, _D), jnp.float32))
    return pl.pallas_call(
        _tc_body, grid=grid, in_specs=in_specs, out_specs=out_specs,
        out_shape=out_shape,
    )(xb, sums, degs, weo, rf8, W_O, b_O, W_I, b_I, W_S, b_S, W_R, b_R)


def kernel(x, edge_index, r_feats, W_O, b_O, W_I, b_I, W_S, b_S, W_R, b_R):
    xb = x.astype(jnp.bfloat16)
    x0 = xb[:, :_DH]
    x1 = xb[:, _DH:]
    eq = edge_index.reshape(2, _E // _C, _C)
    srcq = eq[0]
    dstq = eq[1]

    sums, degs = _sc_segment_sums(x0, x1, srcq, dstq)

    weo = jnp.stack([W_O[:, :_DH], W_O[:, _DH:], W_I[:, :_DH], W_I[:, _DH:]])

    rf8 = jnp.pad(r_feats, ((0, 5), (0, 0)))
    n_out, r8 = _tc_combine(
        xb, sums, degs, weo, rf8,
        W_O, b_O.reshape(1, _D), W_I, b_I.reshape(1, _D),
        W_S, b_S.reshape(1, _D), W_R, b_R.reshape(1, _D))
    return (n_out, r8[:3])
